# baseline (speedup 1.0000x reference)
import jax
import jax.numpy as jnp
from jax.experimental import pallas as pl

B = 16
HW = 320
PATCH = 8
D_FEAT = 384
N_CLUSTER = 27
N_POS = 128
TEMP = 0.1


def _patch_embed(img, W):
    b, c, h, w = img.shape
    ph, pw = h // PATCH, w // PATCH
    x = img.reshape(b, c, ph, PATCH, pw, PATCH)
    x = x.transpose(0, 2, 4, 1, 3, 5).reshape(b * ph * pw, c * PATCH * PATCH)
    return x @ W


def kernel(img, scale, offset, W_embed, centroids, W1_0, W2_0, W1_1, W2_1):
    img_aug = img * scale + offset
    ori_feat = _patch_embed(img, W_embed)
    aug_feat = _patch_embed(img_aug, W_embed)
    c2 = jnp.sum(centroids ** 2, axis=1, keepdims=True)
    x2 = jnp.sum(ori_feat ** 2, axis=1)[None, :]
    d2 = c2 + x2 - 2.0 * (centroids @ ori_feat.T)
    dist = jnp.sqrt(jnp.clip(d2, 0.0))
    _, idx = jax.lax.top_k(dist, N_POS)
    flat_idx = idx.reshape(-1)
    pos_ori_feat = jnp.take(ori_feat, flat_idx, axis=0)
    pos_aug_feat = jnp.take(aug_feat, flat_idx, axis=0)
    dino_feat = jnp.concatenate([pos_ori_feat, pos_aug_feat], axis=0)
    h = dino_feat
    for W1, W2 in ((W1_0, W2_0), (W1_1, W2_1)):
        h = h + jax.nn.relu(h @ W1) @ W2
    semantic_feat_img1, semantic_feat_img2 = jnp.split(h, 2, axis=0)
    f1 = semantic_feat_img1 / (jnp.linalg.norm(semantic_feat_img1, axis=1, keepdims=True) + 1e-8)
    f2 = semantic_feat_img2 / (jnp.linalg.norm(semantic_feat_img2, axis=1, keepdims=True) + 1e-8)
    logits = (f1 @ f2.T) / TEMP
    logp = jax.nn.log_softmax(logits, axis=1)
    loss = -jnp.mean(jnp.diagonal(logp))
    return semantic_feat_img1, loss


# speedup vs baseline: 1.9781x; 1.9781x over previous
import functools

import jax
import jax.numpy as jnp
from jax import lax
from jax.experimental import pallas as pl
from jax.experimental.pallas import tpu as pltpu
from jax.experimental.pallas import tpu_sc as plsc

B = 16
HW = 320
PATCH = 8
D_PATCH = 192
D_PAD = 256
D_FEAT = 384
N_CLUSTER = 27
N_POS = 128
TEMP = 0.1
N_TOK = 25600
TN = 512


def _s1a_body(x_ref, w_ref, c_ref, f_ref, cf_ref):
    f = jnp.dot(x_ref[...], w_ref[...], preferred_element_type=jnp.float32)
    f_ref[...] = f
    cf_ref[...] = jax.lax.dot_general(c_ref[...], f, (((1,), (1,)), ((), ())),
                                      preferred_element_type=jnp.float32)


def _s1a(x, W_embed, centroids):
    grid = N_TOK // TN
    return pl.pallas_call(
        _s1a_body,
        grid=(grid,),
        in_specs=[
            pl.BlockSpec((TN, D_PAD), lambda i: (i, 0)),
            pl.BlockSpec((D_PAD, D_FEAT), lambda i: (0, 0)),
            pl.BlockSpec((N_CLUSTER, D_FEAT), lambda i: (0, 0)),
        ],
        out_specs=[
            pl.BlockSpec((TN, D_FEAT), lambda i: (i, 0)),
            pl.BlockSpec((N_CLUSTER, TN), lambda i: (0, i)),
        ],
        out_shape=[
            jax.ShapeDtypeStruct((N_TOK, D_FEAT), jnp.float32),
            jax.ShapeDtypeStruct((N_CLUSTER, N_TOK), jnp.float32),
        ],
    )(x, W_embed, centroids)


def _s1b_body(cf_ref, x2_ref, c2_ref, dist_ref):
    d2 = c2_ref[...] + x2_ref[...] - 2.0 * cf_ref[...]
    dist_ref[...] = jnp.sqrt(jnp.clip(d2, 0.0))


def _s1b(cf, x2, c2):
    grid = N_TOK // 3200
    return pl.pallas_call(
        _s1b_body,
        grid=(grid,),
        in_specs=[
            pl.BlockSpec((N_CLUSTER, 3200), lambda i: (0, i)),
            pl.BlockSpec((1, 3200), lambda i: (0, i)),
            pl.BlockSpec((N_CLUSTER, 1), lambda i: (0, 0)),
        ],
        out_specs=pl.BlockSpec((N_CLUSTER, 3200), lambda i: (0, i)),
        out_shape=jax.ShapeDtypeStruct((N_CLUSTER, N_TOK), jnp.float32),
    )(cf, x2, c2)


_CHUNK = 256
_NCHUNK = N_TOK // _CHUNK
_L1PAD = 112
_BIG = 1 << 30


def _s2_topk_gather(dist, x):
    mesh = plsc.VectorSubcoreMesh(core_axis_name="c", subcore_axis_name="s")

    @functools.partial(
        pl.kernel, mesh=mesh,
        compiler_params=pltpu.CompilerParams(needs_layout_passes=False),
        out_type=[
            jax.ShapeDtypeStruct((N_CLUSTER, N_POS), jnp.int32),
            jax.ShapeDtypeStruct((N_CLUSTER * N_POS, D_PAD), jnp.float32),
        ],
        scratch_types=[
            pltpu.VMEM((N_TOK,), jnp.float32),
            pltpu.VMEM((_L1PAD,), jnp.float32),
            pltpu.VMEM((N_POS,), jnp.int32),
            pltpu.VMEM((N_POS, D_PAD), jnp.float32),
            pltpu.SemaphoreType.DMA,
        ],
    )
    def k(dist_hbm, x_hbm, idx_hbm, posx_hbm, row_v, l1_v, idxb_v, rows_v, sem):
        wid = lax.axis_index("s") * 2 + lax.axis_index("c")

        @pl.when(wid < N_CLUSTER)
        def _():
            iota = lax.iota(jnp.int32, 16)
            pltpu.sync_copy(dist_hbm.at[wid], row_v)

            l1_v[pl.ds(96, 16)] = jnp.where(iota < 4, 0.0, -1.0)

            def build_chunk(c, carry):
                base = c * _CHUNK
                m = row_v[pl.ds(base, 16)]
                for j in range(1, 16):
                    m = jnp.maximum(m, row_v[pl.ds(base + j * 16, 16)])
                cm = jnp.max(m)
                off = (c // 16) * 16
                cur = l1_v[pl.ds(off, 16)]
                l1_v[pl.ds(off, 16)] = jnp.where(iota == (c - off), cm, cur)
                return carry

            lax.fori_loop(0, _NCHUNK, build_chunk, 0)

            def extract(i, carry):
                v = l1_v[pl.ds(0, 16)]
                for g in range(1, 7):
                    v = jnp.maximum(v, l1_v[pl.ds(g * 16, 16)])
                m = jnp.max(v)
                cstar = _BIG
                for g in range(7):
                    lv = l1_v[pl.ds(g * 16, 16)]
                    cand = jnp.where(lv == m, iota + g * 16, _BIG)
                    cstar = jnp.minimum(cstar, jnp.min(cand))
                base = cstar * _CHUNK
                gidx = _BIG
                for j in range(16):
                    vj = row_v[pl.ds(base + j * 16, 16)]
                    cand = jnp.where(vj == m, iota + j * 16, _BIG)
                    gidx = jnp.minimum(gidx, jnp.min(cand))
                gflat = base + gidx
                goff = (i // 16) * 16
                ib = idxb_v[pl.ds(goff, 16)]
                idxb_v[pl.ds(goff, 16)] = jnp.where(iota == (i - goff), gflat, ib)
                ebase = (gflat // 16) * 16
                ev = row_v[pl.ds(ebase, 16)]
                row_v[pl.ds(ebase, 16)] = jnp.where(iota == (gflat - ebase), -1.0, ev)
                m2 = row_v[pl.ds(base, 16)]
                for j in range(1, 16):
                    m2 = jnp.maximum(m2, row_v[pl.ds(base + j * 16, 16)])
                cm2 = jnp.max(m2)
                loff = (cstar // 16) * 16
                lv2 = l1_v[pl.ds(loff, 16)]
                l1_v[pl.ds(loff, 16)] = jnp.where(iota == (cstar - loff), cm2, lv2)
                return carry

            lax.fori_loop(0, N_POS, extract, 0)

            pltpu.async_copy(x_hbm.at[idxb_v], rows_v, sem).wait()
            pltpu.sync_copy(rows_v, posx_hbm.at[pl.ds(wid * N_POS, N_POS)])
            pltpu.sync_copy(idxb_v, idx_hbm.at[wid])

    return k(dist, x)


def kernel(img, scale, offset, W_embed, centroids, W1_0, W2_0, W1_1, W2_1):
    x = img.reshape(B, 3, HW // PATCH, PATCH, HW // PATCH, PATCH)
    x = x.transpose(0, 2, 4, 1, 3, 5).reshape(N_TOK, D_PATCH)
    x = jnp.pad(x, ((0, 0), (0, D_PAD - D_PATCH)))
    W_pad = jnp.pad(W_embed, ((0, D_PAD - D_PATCH), (0, 0)))

    f, cf = _s1a(x, W_pad, centroids)
    x2 = jnp.sum(f ** 2, axis=1)[None, :]
    c2 = jnp.sum(centroids ** 2, axis=1, keepdims=True)
    dist = _s1b(cf, x2, c2)

    idx, pos_x = _s2_topk_gather(dist, x)
    flat_idx = idx.reshape(-1)
    b_idx = flat_idx // (N_TOK // B)
    s_tab = jnp.repeat(scale.reshape(B, 3), PATCH * PATCH, axis=1)
    o_tab = jnp.repeat(offset.reshape(B, 3), PATCH * PATCH, axis=1)
    s_tab = jnp.pad(s_tab, ((0, 0), (0, D_PAD - D_PATCH)))
    o_tab = jnp.pad(o_tab, ((0, 0), (0, D_PAD - D_PATCH)))
    o_vec = o_tab @ W_pad
    pos_ori = pos_x @ W_pad
    pos_aug = (pos_x * s_tab[b_idx]) @ W_pad + o_vec[b_idx]

    h = jnp.concatenate([pos_ori, pos_aug], axis=0)
    for W1, W2 in ((W1_0, W2_0), (W1_1, W2_1)):
        h = h + jax.nn.relu(h @ W1) @ W2
    f1h, f2h = jnp.split(h, 2, axis=0)
    f1 = f1h / (jnp.linalg.norm(f1h, axis=1, keepdims=True) + 1e-8)
    f2 = f2h / (jnp.linalg.norm(f2h, axis=1, keepdims=True) + 1e-8)
    logits = (f1 @ f2.T) / TEMP
    logp = jax.nn.log_softmax(logits, axis=1)
    loss = -jnp.mean(jnp.diagonal(logp))
    return f1h, loss


# speedup vs baseline: 2.4577x; 1.2424x over previous
import functools

import jax
import jax.numpy as jnp
from jax import lax
from jax.experimental import pallas as pl
from jax.experimental.pallas import tpu as pltpu
from jax.experimental.pallas import tpu_sc as plsc

B = 16
HW = 320
PATCH = 8
D_PATCH = 192
D_PAD = 256
D_FEAT = 384
N_CLUSTER = 27
N_POS = 128
TEMP = 0.1
N_TOK = 25600
TN = 512


def _s1a_body(x_ref, w_ref, c_ref, f_ref, cf_ref):
    f = jnp.dot(x_ref[...], w_ref[...], preferred_element_type=jnp.float32)
    f_ref[...] = f
    cf_ref[...] = jax.lax.dot_general(c_ref[...], f, (((1,), (1,)), ((), ())),
                                      preferred_element_type=jnp.float32)


def _s1a(x, W_embed, centroids):
    grid = N_TOK // TN
    return pl.pallas_call(
        _s1a_body,
        grid=(grid,),
        in_specs=[
            pl.BlockSpec((TN, D_PAD), lambda i: (i, 0)),
            pl.BlockSpec((D_PAD, D_FEAT), lambda i: (0, 0)),
            pl.BlockSpec((N_CLUSTER, D_FEAT), lambda i: (0, 0)),
        ],
        out_specs=[
            pl.BlockSpec((TN, D_FEAT), lambda i: (i, 0)),
            pl.BlockSpec((N_CLUSTER, TN), lambda i: (0, i)),
        ],
        out_shape=[
            jax.ShapeDtypeStruct((N_TOK, D_FEAT), jnp.float32),
            jax.ShapeDtypeStruct((N_CLUSTER, N_TOK), jnp.float32),
        ],
    )(x, W_embed, centroids)


def _s1b_body(cf_ref, x2_ref, c2_ref, dist_ref):
    d2 = c2_ref[...] + x2_ref[...] - 2.0 * cf_ref[...]
    dist_ref[...] = jnp.sqrt(jnp.clip(d2, 0.0))


def _s1b(cf, x2, c2):
    grid = N_TOK // 3200
    return pl.pallas_call(
        _s1b_body,
        grid=(grid,),
        in_specs=[
            pl.BlockSpec((N_CLUSTER, 3200), lambda i: (0, i)),
            pl.BlockSpec((1, 3200), lambda i: (0, i)),
            pl.BlockSpec((N_CLUSTER, 1), lambda i: (0, 0)),
        ],
        out_specs=pl.BlockSpec((N_CLUSTER, 3200), lambda i: (0, i)),
        out_shape=jax.ShapeDtypeStruct((N_CLUSTER, N_TOK), jnp.float32),
    )(cf, x2, c2)


_CHUNK = 256
_NCHUNK = N_TOK // _CHUNK
_L1PAD = 112
_BIG = 1 << 30


def _s2_topk_gather(dist, x):
    mesh = plsc.VectorSubcoreMesh(core_axis_name="c", subcore_axis_name="s")

    @functools.partial(
        pl.kernel, mesh=mesh,
        compiler_params=pltpu.CompilerParams(needs_layout_passes=False),
        out_type=[
            jax.ShapeDtypeStruct((N_CLUSTER, N_POS), jnp.int32),
            jax.ShapeDtypeStruct((N_CLUSTER * N_POS, D_PAD), jnp.float32),
        ],
        scratch_types=[
            pltpu.VMEM((N_TOK,), jnp.float32),
            pltpu.VMEM((_L1PAD,), jnp.float32),
            pltpu.VMEM((N_POS,), jnp.int32),
            pltpu.VMEM((N_POS, D_PAD), jnp.float32),
            pltpu.SemaphoreType.DMA,
        ],
    )
    def k(dist_hbm, x_hbm, idx_hbm, posx_hbm, row_v, l1_v, idxb_v, rows_v, sem):
        wid = lax.axis_index("s") * 2 + lax.axis_index("c")

        @pl.when(wid < N_CLUSTER)
        def _():
            iota = lax.iota(jnp.int32, 16)
            pltpu.sync_copy(dist_hbm.at[wid], row_v)

            l1_v[pl.ds(96, 16)] = jnp.where(iota < 4, 0.0, -1.0)

            def build_chunk(c, carry):
                base = c * _CHUNK
                m = row_v[pl.ds(base, 16)]
                for j in range(1, 16):
                    m = jnp.maximum(m, row_v[pl.ds(base + j * 16, 16)])
                cm = jnp.max(m)
                off = (c // 16) * 16
                cur = l1_v[pl.ds(off, 16)]
                l1_v[pl.ds(off, 16)] = jnp.where(iota == (c - off), cm, cur)
                return carry

            lax.fori_loop(0, _NCHUNK, build_chunk, 0)

            def extract(i, carry):
                v = l1_v[pl.ds(0, 16)]
                for g in range(1, 7):
                    v = jnp.maximum(v, l1_v[pl.ds(g * 16, 16)])
                m = jnp.max(v)
                cstar = _BIG
                for g in range(7):
                    lv = l1_v[pl.ds(g * 16, 16)]
                    cand = jnp.where(lv == m, iota + g * 16, _BIG)
                    cstar = jnp.minimum(cstar, jnp.min(cand))
                base = cstar * _CHUNK
                gidx = _BIG
                for j in range(16):
                    vj = row_v[pl.ds(base + j * 16, 16)]
                    cand = jnp.where(vj == m, iota + j * 16, _BIG)
                    gidx = jnp.minimum(gidx, jnp.min(cand))
                gflat = base + gidx
                goff = (i // 16) * 16
                ib = idxb_v[pl.ds(goff, 16)]
                idxb_v[pl.ds(goff, 16)] = jnp.where(iota == (i - goff), gflat, ib)
                ebase = (gflat // 16) * 16
                ev = row_v[pl.ds(ebase, 16)]
                row_v[pl.ds(ebase, 16)] = jnp.where(iota == (gflat - ebase), -1.0, ev)
                m2 = row_v[pl.ds(base, 16)]
                for j in range(1, 16):
                    m2 = jnp.maximum(m2, row_v[pl.ds(base + j * 16, 16)])
                cm2 = jnp.max(m2)
                loff = (cstar // 16) * 16
                lv2 = l1_v[pl.ds(loff, 16)]
                l1_v[pl.ds(loff, 16)] = jnp.where(iota == (cstar - loff), cm2, lv2)
                return carry

            lax.fori_loop(0, N_POS, extract, 0)

            pltpu.async_copy(x_hbm.at[idxb_v], rows_v, sem).wait()
            pltpu.sync_copy(rows_v, posx_hbm.at[pl.ds(wid * N_POS, N_POS)])
            pltpu.sync_copy(idxb_v, idx_hbm.at[wid])

    return k(dist, x)


_TS3 = 384


def _s3_body(px_ref, bi_ref, stab_ref, otab_ref, w_ref, w10_ref, w20_ref,
             w11_ref, w21_ref, out1_ref, f1n_ref, f2n_ref):
    px = px_ref[...]
    f_ori = jnp.dot(px, w_ref[...], preferred_element_type=jnp.float32)
    onehot = (bi_ref[...] == lax.broadcasted_iota(jnp.int32, (1, B), 1)
              ).astype(jnp.float32)
    s_g = jnp.dot(onehot, stab_ref[...], preferred_element_type=jnp.float32)
    o_g = jnp.dot(onehot,
                  jnp.dot(otab_ref[...], w_ref[...],
                          preferred_element_type=jnp.float32),
                  preferred_element_type=jnp.float32)
    f_aug = jnp.dot(px * s_g, w_ref[...], preferred_element_type=jnp.float32) + o_g

    def mlp(h):
        for w1_ref, w2_ref in ((w10_ref, w20_ref), (w11_ref, w21_ref)):
            t = jax.nn.relu(jnp.dot(h, w1_ref[...],
                                    preferred_element_type=jnp.float32))
            h = h + jnp.dot(t, w2_ref[...], preferred_element_type=jnp.float32)
        return h

    h_o = mlp(f_ori)
    h_a = mlp(f_aug)
    out1_ref[...] = h_o
    f1n_ref[...] = h_o / (jnp.sqrt(jnp.sum(h_o * h_o, axis=1, keepdims=True)) + 1e-8)
    f2n_ref[...] = h_a / (jnp.sqrt(jnp.sum(h_a * h_a, axis=1, keepdims=True)) + 1e-8)


def _s3(pos_x, b_idx, s_tab, o_tab, W_pad, W1_0, W2_0, W1_1, W2_1):
    n = N_CLUSTER * N_POS
    grid = n // _TS3
    full = lambda r, c: pl.BlockSpec((r, c), lambda i: (0, 0))
    return pl.pallas_call(
        _s3_body,
        grid=(grid,),
        in_specs=[
            pl.BlockSpec((_TS3, D_PAD), lambda i: (i, 0)),
            pl.BlockSpec((_TS3, 1), lambda i: (i, 0)),
            full(B, D_PAD), full(B, D_PAD), full(D_PAD, D_FEAT),
            full(D_FEAT, D_FEAT), full(D_FEAT, D_FEAT),
            full(D_FEAT, D_FEAT), full(D_FEAT, D_FEAT),
        ],
        out_specs=[
            pl.BlockSpec((_TS3, D_FEAT), lambda i: (i, 0)),
            pl.BlockSpec((_TS3, D_FEAT), lambda i: (i, 0)),
            pl.BlockSpec((_TS3, D_FEAT), lambda i: (i, 0)),
        ],
        out_shape=[
            jax.ShapeDtypeStruct((n, D_FEAT), jnp.float32),
            jax.ShapeDtypeStruct((n, D_FEAT), jnp.float32),
            jax.ShapeDtypeStruct((n, D_FEAT), jnp.float32),
        ],
    )(pos_x, b_idx, s_tab, o_tab, W_pad, W1_0, W2_0, W1_1, W2_1)


def _s4_body(f1_ref, f2_ref, loss_ref, acc_ref):
    i = pl.program_id(0)
    logits = jax.lax.dot_general(
        f1_ref[...], f2_ref[...], (((1,), (1,)), ((), ())),
        preferred_element_type=jnp.float32) / TEMP
    m = jnp.max(logits, axis=1, keepdims=True)
    lse = m + jnp.log(jnp.sum(jnp.exp(logits - m), axis=1, keepdims=True))
    rows = lax.broadcasted_iota(jnp.int32, (_TS3, 1), 0)
    cols = lax.broadcasted_iota(jnp.int32, (1, N_CLUSTER * N_POS), 1)
    diag = jnp.sum(jnp.where(cols == rows + i * _TS3, logits, 0.0),
                   axis=1, keepdims=True)
    part = jnp.sum(lse - diag)

    @pl.when(i == 0)
    def _():
        acc_ref[0, 0] = 0.0

    acc_ref[0, 0] += part

    @pl.when(i == pl.num_programs(0) - 1)
    def _():
        loss_ref[...] = jnp.full((1, 1), acc_ref[0, 0] / (N_CLUSTER * N_POS),
                                 jnp.float32)


def _s4(f1n, f2n):
    n = N_CLUSTER * N_POS
    grid = n // _TS3
    return pl.pallas_call(
        _s4_body,
        grid=(grid,),
        in_specs=[
            pl.BlockSpec((_TS3, D_FEAT), lambda i: (i, 0)),
            pl.BlockSpec((n, D_FEAT), lambda i: (0, 0)),
        ],
        out_specs=pl.BlockSpec((1, 1), lambda i: (0, 0)),
        out_shape=jax.ShapeDtypeStruct((1, 1), jnp.float32),
        scratch_shapes=[pltpu.SMEM((1, 1), jnp.float32)],
    )(f1n, f2n)


def kernel(img, scale, offset, W_embed, centroids, W1_0, W2_0, W1_1, W2_1):
    x = img.reshape(B, 3, HW // PATCH, PATCH, HW // PATCH, PATCH)
    x = x.transpose(0, 2, 4, 1, 3, 5).reshape(N_TOK, D_PATCH)
    x = jnp.pad(x, ((0, 0), (0, D_PAD - D_PATCH)))
    W_pad = jnp.pad(W_embed, ((0, D_PAD - D_PATCH), (0, 0)))

    f, cf = _s1a(x, W_pad, centroids)
    x2 = jnp.sum(f ** 2, axis=1)[None, :]
    c2 = jnp.sum(centroids ** 2, axis=1, keepdims=True)
    dist = _s1b(cf, x2, c2)

    idx, pos_x = _s2_topk_gather(dist, x)
    b_idx = (idx.reshape(-1, 1) // (N_TOK // B)).astype(jnp.int32)
    s_tab = jnp.repeat(scale.reshape(B, 3), PATCH * PATCH, axis=1)
    o_tab = jnp.repeat(offset.reshape(B, 3), PATCH * PATCH, axis=1)
    s_tab = jnp.pad(s_tab, ((0, 0), (0, D_PAD - D_PATCH)))
    o_tab = jnp.pad(o_tab, ((0, 0), (0, D_PAD - D_PATCH)))

    out1, f1n, f2n = _s3(pos_x, b_idx, s_tab, o_tab, W_pad,
                         W1_0, W2_0, W1_1, W2_1)
    loss = _s4(f1n, f2n).reshape(())
    return out1, loss


# speedup vs baseline: 2.4578x; 1.0001x over previous
import functools

import jax
import jax.numpy as jnp
from jax import lax
from jax.experimental import pallas as pl
from jax.experimental.pallas import tpu as pltpu
from jax.experimental.pallas import tpu_sc as plsc

B = 16
HW = 320
PATCH = 8
D_PATCH = 192
D_PAD = 256
D_FEAT = 384
N_CLUSTER = 27
N_POS = 128
TEMP = 0.1
N_TOK = 25600
TN = 512


def _s1a_body(x_ref, w_ref, c_ref, f_ref, cf_ref):
    f = jnp.dot(x_ref[...], w_ref[...], preferred_element_type=jnp.float32)
    f_ref[...] = f
    cf_ref[...] = jax.lax.dot_general(c_ref[...], f, (((1,), (1,)), ((), ())),
                                      preferred_element_type=jnp.float32)


def _s1a(x, W_embed, centroids):
    grid = N_TOK // TN
    return pl.pallas_call(
        _s1a_body,
        grid=(grid,),
        in_specs=[
            pl.BlockSpec((TN, D_PAD), lambda i: (i, 0)),
            pl.BlockSpec((D_PAD, D_FEAT), lambda i: (0, 0)),
            pl.BlockSpec((N_CLUSTER, D_FEAT), lambda i: (0, 0)),
        ],
        out_specs=[
            pl.BlockSpec((TN, D_FEAT), lambda i: (i, 0)),
            pl.BlockSpec((N_CLUSTER, TN), lambda i: (0, i)),
        ],
        out_shape=[
            jax.ShapeDtypeStruct((N_TOK, D_FEAT), jnp.float32),
            jax.ShapeDtypeStruct((N_CLUSTER, N_TOK), jnp.float32),
        ],
    )(x, W_embed, centroids)


def _s1b_body(cf_ref, x2_ref, c2_ref, dist_ref):
    d2 = c2_ref[...] + x2_ref[...] - 2.0 * cf_ref[...]
    dist_ref[...] = jnp.sqrt(jnp.clip(d2, 0.0))


def _s1b(cf, x2, c2):
    grid = N_TOK // 3200
    return pl.pallas_call(
        _s1b_body,
        grid=(grid,),
        in_specs=[
            pl.BlockSpec((N_CLUSTER, 3200), lambda i: (0, i)),
            pl.BlockSpec((1, 3200), lambda i: (0, i)),
            pl.BlockSpec((N_CLUSTER, 1), lambda i: (0, 0)),
        ],
        out_specs=pl.BlockSpec((N_CLUSTER, 3200), lambda i: (0, i)),
        out_shape=jax.ShapeDtypeStruct((N_CLUSTER, N_TOK), jnp.float32),
    )(cf, x2, c2)


_CHUNK = 256
_NCHUNK = N_TOK // _CHUNK
_L1PAD = 112
_BIG = 1 << 30


def _s2_topk_gather(dist, x):
    mesh = plsc.VectorSubcoreMesh(core_axis_name="c", subcore_axis_name="s")

    @functools.partial(
        pl.kernel, mesh=mesh,
        compiler_params=pltpu.CompilerParams(needs_layout_passes=False,
                                             use_tc_tiling_on_sc=True),
        out_type=[
            jax.ShapeDtypeStruct((N_CLUSTER, N_POS), jnp.int32),
            jax.ShapeDtypeStruct((N_CLUSTER * N_POS, D_PAD), jnp.float32),
        ],
        scratch_types=[
            pltpu.VMEM((N_TOK,), jnp.float32),
            pltpu.VMEM((_L1PAD,), jnp.float32),
            pltpu.VMEM((N_POS,), jnp.int32),
            pltpu.VMEM((N_POS, D_PAD), jnp.float32),
            pltpu.SemaphoreType.DMA,
        ],
    )
    def k(dist_hbm, x_hbm, idx_hbm, posx_hbm, row_v, l1_v, idxb_v, rows_v, sem):
        wid = lax.axis_index("s") * 2 + lax.axis_index("c")

        @pl.when(wid < N_CLUSTER)
        def _():
            iota = lax.iota(jnp.int32, 16)
            pltpu.sync_copy(dist_hbm.at[wid], row_v)

            l1_v[pl.ds(96, 16)] = jnp.where(iota < 4, 0.0, -1.0)

            def build_chunk(c, carry):
                base = c * _CHUNK
                m = row_v[pl.ds(base, 16)]
                for j in range(1, 16):
                    m = jnp.maximum(m, row_v[pl.ds(base + j * 16, 16)])
                cm = jnp.max(m)
                off = (c // 16) * 16
                cur = l1_v[pl.ds(off, 16)]
                l1_v[pl.ds(off, 16)] = jnp.where(iota == (c - off), cm, cur)
                return carry

            lax.fori_loop(0, _NCHUNK, build_chunk, 0)

            def extract(i, carry):
                v = l1_v[pl.ds(0, 16)]
                for g in range(1, 7):
                    v = jnp.maximum(v, l1_v[pl.ds(g * 16, 16)])
                m = jnp.max(v)
                cstar = _BIG
                for g in range(7):
                    lv = l1_v[pl.ds(g * 16, 16)]
                    cand = jnp.where(lv == m, iota + g * 16, _BIG)
                    cstar = jnp.minimum(cstar, jnp.min(cand))
                base = cstar * _CHUNK
                gidx = _BIG
                for j in range(16):
                    vj = row_v[pl.ds(base + j * 16, 16)]
                    cand = jnp.where(vj == m, iota + j * 16, _BIG)
                    gidx = jnp.minimum(gidx, jnp.min(cand))
                gflat = base + gidx
                goff = (i // 16) * 16
                ib = idxb_v[pl.ds(goff, 16)]
                idxb_v[pl.ds(goff, 16)] = jnp.where(iota == (i - goff), gflat, ib)
                ebase = (gflat // 16) * 16
                ev = row_v[pl.ds(ebase, 16)]
                row_v[pl.ds(ebase, 16)] = jnp.where(iota == (gflat - ebase), -1.0, ev)
                m2 = row_v[pl.ds(base, 16)]
                for j in range(1, 16):
                    m2 = jnp.maximum(m2, row_v[pl.ds(base + j * 16, 16)])
                cm2 = jnp.max(m2)
                loff = (cstar // 16) * 16
                lv2 = l1_v[pl.ds(loff, 16)]
                l1_v[pl.ds(loff, 16)] = jnp.where(iota == (cstar - loff), cm2, lv2)
                return carry

            lax.fori_loop(0, N_POS, extract, 0)

            pltpu.async_copy(x_hbm.at[idxb_v], rows_v, sem).wait()
            pltpu.sync_copy(rows_v, posx_hbm.at[pl.ds(wid * N_POS, N_POS)])
            pltpu.sync_copy(idxb_v, idx_hbm.at[wid])

    return k(dist, x)


_TS3 = 384


def _s3_body(px_ref, bi_ref, stab_ref, otab_ref, w_ref, w10_ref, w20_ref,
             w11_ref, w21_ref, out1_ref, f1n_ref, f2n_ref):
    px = px_ref[...]
    f_ori = jnp.dot(px, w_ref[...], preferred_element_type=jnp.float32)
    onehot = (bi_ref[...] == lax.broadcasted_iota(jnp.int32, (1, B), 1)
              ).astype(jnp.float32)
    s_g = jnp.dot(onehot, stab_ref[...], preferred_element_type=jnp.float32)
    o_g = jnp.dot(onehot,
                  jnp.dot(otab_ref[...], w_ref[...],
                          preferred_element_type=jnp.float32),
                  preferred_element_type=jnp.float32)
    f_aug = jnp.dot(px * s_g, w_ref[...], preferred_element_type=jnp.float32) + o_g

    def mlp(h):
        for w1_ref, w2_ref in ((w10_ref, w20_ref), (w11_ref, w21_ref)):
            t = jax.nn.relu(jnp.dot(h, w1_ref[...],
                                    preferred_element_type=jnp.float32))
            h = h + jnp.dot(t, w2_ref[...], preferred_element_type=jnp.float32)
        return h

    h_o = mlp(f_ori)
    h_a = mlp(f_aug)
    out1_ref[...] = h_o
    f1n_ref[...] = h_o / (jnp.sqrt(jnp.sum(h_o * h_o, axis=1, keepdims=True)) + 1e-8)
    f2n_ref[...] = h_a / (jnp.sqrt(jnp.sum(h_a * h_a, axis=1, keepdims=True)) + 1e-8)


def _s3(pos_x, b_idx, s_tab, o_tab, W_pad, W1_0, W2_0, W1_1, W2_1):
    n = N_CLUSTER * N_POS
    grid = n // _TS3
    full = lambda r, c: pl.BlockSpec((r, c), lambda i: (0, 0))
    return pl.pallas_call(
        _s3_body,
        grid=(grid,),
        in_specs=[
            pl.BlockSpec((_TS3, D_PAD), lambda i: (i, 0)),
            pl.BlockSpec((_TS3, 1), lambda i: (i, 0)),
            full(B, D_PAD), full(B, D_PAD), full(D_PAD, D_FEAT),
            full(D_FEAT, D_FEAT), full(D_FEAT, D_FEAT),
            full(D_FEAT, D_FEAT), full(D_FEAT, D_FEAT),
        ],
        out_specs=[
            pl.BlockSpec((_TS3, D_FEAT), lambda i: (i, 0)),
            pl.BlockSpec((_TS3, D_FEAT), lambda i: (i, 0)),
            pl.BlockSpec((_TS3, D_FEAT), lambda i: (i, 0)),
        ],
        out_shape=[
            jax.ShapeDtypeStruct((n, D_FEAT), jnp.float32),
            jax.ShapeDtypeStruct((n, D_FEAT), jnp.float32),
            jax.ShapeDtypeStruct((n, D_FEAT), jnp.float32),
        ],
    )(pos_x, b_idx, s_tab, o_tab, W_pad, W1_0, W2_0, W1_1, W2_1)


def _s4_body(f1_ref, f2_ref, loss_ref, acc_ref):
    i = pl.program_id(0)
    logits = jax.lax.dot_general(
        f1_ref[...], f2_ref[...], (((1,), (1,)), ((), ())),
        preferred_element_type=jnp.float32) / TEMP
    m = jnp.max(logits, axis=1, keepdims=True)
    lse = m + jnp.log(jnp.sum(jnp.exp(logits - m), axis=1, keepdims=True))
    rows = lax.broadcasted_iota(jnp.int32, (_TS3, 1), 0)
    cols = lax.broadcasted_iota(jnp.int32, (1, N_CLUSTER * N_POS), 1)
    diag = jnp.sum(jnp.where(cols == rows + i * _TS3, logits, 0.0),
                   axis=1, keepdims=True)
    part = jnp.sum(lse - diag)

    @pl.when(i == 0)
    def _():
        acc_ref[0, 0] = 0.0

    acc_ref[0, 0] += part

    @pl.when(i == pl.num_programs(0) - 1)
    def _():
        loss_ref[...] = jnp.full((1, 1), acc_ref[0, 0] / (N_CLUSTER * N_POS),
                                 jnp.float32)


def _s4(f1n, f2n):
    n = N_CLUSTER * N_POS
    grid = n // _TS3
    return pl.pallas_call(
        _s4_body,
        grid=(grid,),
        in_specs=[
            pl.BlockSpec((_TS3, D_FEAT), lambda i: (i, 0)),
            pl.BlockSpec((n, D_FEAT), lambda i: (0, 0)),
        ],
        out_specs=pl.BlockSpec((1, 1), lambda i: (0, 0)),
        out_shape=jax.ShapeDtypeStruct((1, 1), jnp.float32),
        scratch_shapes=[pltpu.SMEM((1, 1), jnp.float32)],
    )(f1n, f2n)


def kernel(img, scale, offset, W_embed, centroids, W1_0, W2_0, W1_1, W2_1):
    x = img.reshape(B, 3, HW // PATCH, PATCH, HW // PATCH, PATCH)
    x = x.transpose(0, 2, 4, 1, 3, 5).reshape(N_TOK, D_PATCH)
    x = jnp.pad(x, ((0, 0), (0, D_PAD - D_PATCH)))
    W_pad = jnp.pad(W_embed, ((0, D_PAD - D_PATCH), (0, 0)))

    f, cf = _s1a(x, W_pad, centroids)
    x2 = jnp.sum(f ** 2, axis=1)[None, :]
    c2 = jnp.sum(centroids ** 2, axis=1, keepdims=True)
    dist = _s1b(cf, x2, c2)

    idx, pos_x = _s2_topk_gather(dist, x)
    b_idx = (idx.reshape(-1, 1) // (N_TOK // B)).astype(jnp.int32)
    s_tab = jnp.repeat(scale.reshape(B, 3), PATCH * PATCH, axis=1)
    o_tab = jnp.repeat(offset.reshape(B, 3), PATCH * PATCH, axis=1)
    s_tab = jnp.pad(s_tab, ((0, 0), (0, D_PAD - D_PATCH)))
    o_tab = jnp.pad(o_tab, ((0, 0), (0, D_PAD - D_PATCH)))

    out1, f1n, f2n = _s3(pos_x, b_idx, s_tab, o_tab, W_pad,
                         W1_0, W2_0, W1_1, W2_1)
    loss = _s4(f1n, f2n).reshape(())
    return out1, loss


# speedup vs baseline: 2.4640x; 1.0025x over previous
import functools

import jax
import jax.numpy as jnp
from jax import lax
from jax.experimental import pallas as pl
from jax.experimental.pallas import tpu as pltpu
from jax.experimental.pallas import tpu_sc as plsc

B = 16
HW = 320
PATCH = 8
D_PATCH = 192
D_PAD = 256
D_FEAT = 384
N_CLUSTER = 27
N_POS = 128
TEMP = 0.1
N_TOK = 25600
TN = 512


def _s1a_body(x_ref, w_ref, c_ref, f_ref, cf_ref):
    f = jnp.dot(x_ref[...], w_ref[...], preferred_element_type=jnp.float32)
    f_ref[...] = f
    cf_ref[...] = jax.lax.dot_general(c_ref[...], f, (((1,), (1,)), ((), ())),
                                      preferred_element_type=jnp.float32)


def _s1a(x, W_embed, centroids):
    grid = N_TOK // TN
    return pl.pallas_call(
        _s1a_body,
        grid=(grid,),
        in_specs=[
            pl.BlockSpec((TN, D_PAD), lambda i: (i, 0)),
            pl.BlockSpec((D_PAD, D_FEAT), lambda i: (0, 0)),
            pl.BlockSpec((N_CLUSTER, D_FEAT), lambda i: (0, 0)),
        ],
        out_specs=[
            pl.BlockSpec((TN, D_FEAT), lambda i: (i, 0)),
            pl.BlockSpec((N_CLUSTER, TN), lambda i: (0, i)),
        ],
        out_shape=[
            jax.ShapeDtypeStruct((N_TOK, D_FEAT), jnp.float32),
            jax.ShapeDtypeStruct((N_CLUSTER, N_TOK), jnp.float32),
        ],
    )(x, W_embed, centroids)


def _s1b_body(cf_ref, x2_ref, c2_ref, dist_ref):
    d2 = c2_ref[...] + x2_ref[...] - 2.0 * cf_ref[...]
    dist_ref[...] = jnp.sqrt(jnp.clip(d2, 0.0))


def _s1b(cf, x2, c2):
    grid = N_TOK // 3200
    return pl.pallas_call(
        _s1b_body,
        grid=(grid,),
        in_specs=[
            pl.BlockSpec((N_CLUSTER, 3200), lambda i: (0, i)),
            pl.BlockSpec((1, 3200), lambda i: (0, i)),
            pl.BlockSpec((N_CLUSTER, 1), lambda i: (0, 0)),
        ],
        out_specs=pl.BlockSpec((N_CLUSTER, 3200), lambda i: (0, i)),
        out_shape=jax.ShapeDtypeStruct((N_CLUSTER, N_TOK), jnp.float32),
    )(cf, x2, c2)


_CHUNK = 256
_NCHUNK = N_TOK // _CHUNK
_L1PAD = 112
_BIG = 1 << 30


def _s2_topk_gather(dist, x):
    mesh = plsc.VectorSubcoreMesh(core_axis_name="c", subcore_axis_name="s")

    @functools.partial(
        pl.kernel, mesh=mesh,
        compiler_params=pltpu.CompilerParams(needs_layout_passes=False,
                                             use_tc_tiling_on_sc=True),
        out_type=[
            jax.ShapeDtypeStruct((N_CLUSTER, N_POS), jnp.int32),
            jax.ShapeDtypeStruct((N_CLUSTER * N_POS, D_PAD), jnp.float32),
        ],
        scratch_types=[
            pltpu.VMEM((N_TOK,), jnp.float32),
            pltpu.VMEM((_L1PAD,), jnp.float32),
            pltpu.VMEM((N_POS,), jnp.int32),
            pltpu.VMEM((N_POS, D_PAD), jnp.float32),
            pltpu.SemaphoreType.DMA,
        ],
    )
    def k(dist_hbm, x_hbm, idx_hbm, posx_hbm, row_v, l1_v, idxb_v, rows_v, sem):
        wid = lax.axis_index("s") * 2 + lax.axis_index("c")

        @pl.when(wid < N_CLUSTER)
        def _():
            iota = lax.iota(jnp.int32, 16)
            pltpu.sync_copy(dist_hbm.at[wid], row_v)

            l1_v[pl.ds(96, 16)] = jnp.where(iota < 4, 0.0, -1.0)

            def build_chunk(c, carry):
                base = c * _CHUNK
                m = row_v[pl.ds(base, 16)]
                for j in range(1, 16):
                    m = jnp.maximum(m, row_v[pl.ds(base + j * 16, 16)])
                cm = jnp.max(m)
                off = (c // 16) * 16
                cur = l1_v[pl.ds(off, 16)]
                l1_v[pl.ds(off, 16)] = jnp.where(iota == (c - off), cm, cur)
                return carry

            lax.fori_loop(0, _NCHUNK, build_chunk, 0)

            def extract(i, carry):
                v = l1_v[pl.ds(0, 16)]
                for g in range(1, 7):
                    v = jnp.maximum(v, l1_v[pl.ds(g * 16, 16)])
                m = jnp.max(v)
                cstar = _BIG
                for g in range(7):
                    lv = l1_v[pl.ds(g * 16, 16)]
                    cand = jnp.where(lv == m, iota + g * 16, _BIG)
                    cstar = jnp.minimum(cstar, jnp.min(cand))
                base = cstar * _CHUNK
                gidx = _BIG
                for j in range(16):
                    vj = row_v[pl.ds(base + j * 16, 16)]
                    cand = jnp.where(vj == m, iota + j * 16, _BIG)
                    gidx = jnp.minimum(gidx, jnp.min(cand))
                gflat = base + gidx
                goff = (i // 16) * 16
                ib = idxb_v[pl.ds(goff, 16)]
                idxb_v[pl.ds(goff, 16)] = jnp.where(iota == (i - goff), gflat, ib)
                ebase = (gflat // 16) * 16
                ev = row_v[pl.ds(ebase, 16)]
                row_v[pl.ds(ebase, 16)] = jnp.where(iota == (gflat - ebase), -1.0, ev)
                m2 = row_v[pl.ds(base, 16)]
                for j in range(1, 16):
                    m2 = jnp.maximum(m2, row_v[pl.ds(base + j * 16, 16)])
                cm2 = jnp.max(m2)
                loff = (cstar // 16) * 16
                lv2 = l1_v[pl.ds(loff, 16)]
                l1_v[pl.ds(loff, 16)] = jnp.where(iota == (cstar - loff), cm2, lv2)
                return carry

            lax.fori_loop(0, N_POS, extract, 0)

            pltpu.async_copy(x_hbm.at[idxb_v], rows_v, sem).wait()
            pltpu.sync_copy(rows_v, posx_hbm.at[pl.ds(wid * N_POS, N_POS)])
            pltpu.sync_copy(idxb_v, idx_hbm.at[wid])

    return k(dist, x)


_TS3 = 384


def _s3_body(px_ref, bi_ref, stab_ref, otab_ref, w_ref, w10_ref, w20_ref,
             w11_ref, w21_ref, out1_ref, f1n_ref, f2n_ref):
    px = px_ref[...]
    f_ori = jnp.dot(px, w_ref[...], preferred_element_type=jnp.float32)
    onehot = (bi_ref[...] == lax.broadcasted_iota(jnp.int32, (1, B), 1)
              ).astype(jnp.float32)
    s_g = jnp.dot(onehot, stab_ref[...], preferred_element_type=jnp.float32)
    o_g = jnp.dot(onehot,
                  jnp.dot(otab_ref[...], w_ref[...],
                          preferred_element_type=jnp.float32),
                  preferred_element_type=jnp.float32)
    f_aug = jnp.dot(px * s_g, w_ref[...], preferred_element_type=jnp.float32) + o_g

    def mlp(h):
        for w1_ref, w2_ref in ((w10_ref, w20_ref), (w11_ref, w21_ref)):
            t = jax.nn.relu(jnp.dot(h, w1_ref[...],
                                    preferred_element_type=jnp.float32))
            h = h + jnp.dot(t, w2_ref[...], preferred_element_type=jnp.float32)
        return h

    h_o = mlp(f_ori)
    h_a = mlp(f_aug)
    out1_ref[...] = h_o
    f1n_ref[...] = h_o / (jnp.sqrt(jnp.sum(h_o * h_o, axis=1, keepdims=True)) + 1e-8)
    f2n_ref[...] = h_a / (jnp.sqrt(jnp.sum(h_a * h_a, axis=1, keepdims=True)) + 1e-8)


def _s3(pos_x, b_idx, s_tab, o_tab, W_pad, W1_0, W2_0, W1_1, W2_1):
    n = N_CLUSTER * N_POS
    grid = n // _TS3
    full = lambda r, c: pl.BlockSpec((r, c), lambda i: (0, 0))
    return pl.pallas_call(
        _s3_body,
        grid=(grid,),
        in_specs=[
            pl.BlockSpec((_TS3, D_PAD), lambda i: (i, 0)),
            pl.BlockSpec((_TS3, 1), lambda i: (i, 0)),
            full(B, D_PAD), full(B, D_PAD), full(D_PAD, D_FEAT),
            full(D_FEAT, D_FEAT), full(D_FEAT, D_FEAT),
            full(D_FEAT, D_FEAT), full(D_FEAT, D_FEAT),
        ],
        out_specs=[
            pl.BlockSpec((_TS3, D_FEAT), lambda i: (i, 0)),
            pl.BlockSpec((_TS3, D_FEAT), lambda i: (i, 0)),
            pl.BlockSpec((_TS3, D_FEAT), lambda i: (i, 0)),
        ],
        out_shape=[
            jax.ShapeDtypeStruct((n, D_FEAT), jnp.float32),
            jax.ShapeDtypeStruct((n, D_FEAT), jnp.float32),
            jax.ShapeDtypeStruct((n, D_FEAT), jnp.float32),
        ],
    )(pos_x, b_idx, s_tab, o_tab, W_pad, W1_0, W2_0, W1_1, W2_1)


def _s4_body(f1_ref, f2_ref, f2d_ref, loss_ref, acc_ref):
    i = pl.program_id(0)
    f1 = f1_ref[...]
    logits = jax.lax.dot_general(
        f1, f2_ref[...], (((1,), (1,)), ((), ())),
        preferred_element_type=jnp.float32) / TEMP
    m = jnp.max(logits, axis=1, keepdims=True)
    lse = m + jnp.log(jnp.sum(jnp.exp(logits - m), axis=1, keepdims=True))
    diag = jnp.sum(f1 * f2d_ref[...], axis=1, keepdims=True) / TEMP
    part = jnp.sum(lse - diag)

    @pl.when(i == 0)
    def _():
        acc_ref[0, 0] = 0.0

    acc_ref[0, 0] += part

    @pl.when(i == pl.num_programs(0) - 1)
    def _():
        loss_ref[...] = jnp.full((1, 1), acc_ref[0, 0] / (N_CLUSTER * N_POS),
                                 jnp.float32)


def _s4(f1n, f2n):
    n = N_CLUSTER * N_POS
    grid = n // _TS3
    return pl.pallas_call(
        _s4_body,
        grid=(grid,),
        in_specs=[
            pl.BlockSpec((_TS3, D_FEAT), lambda i: (i, 0)),
            pl.BlockSpec((n, D_FEAT), lambda i: (0, 0)),
            pl.BlockSpec((_TS3, D_FEAT), lambda i: (i, 0)),
        ],
        out_specs=pl.BlockSpec((1, 1), lambda i: (0, 0)),
        out_shape=jax.ShapeDtypeStruct((1, 1), jnp.float32),
        scratch_shapes=[pltpu.SMEM((1, 1), jnp.float32)],
    )(f1n, f2n, f2n)


def kernel(img, scale, offset, W_embed, centroids, W1_0, W2_0, W1_1, W2_1):
    x = img.reshape(B, 3, HW // PATCH, PATCH, HW // PATCH, PATCH)
    x = x.transpose(0, 2, 4, 1, 3, 5).reshape(N_TOK, D_PATCH)
    x = jnp.pad(x, ((0, 0), (0, D_PAD - D_PATCH)))
    W_pad = jnp.pad(W_embed, ((0, D_PAD - D_PATCH), (0, 0)))

    f, cf = _s1a(x, W_pad, centroids)
    x2 = jnp.sum(f ** 2, axis=1)[None, :]
    c2 = jnp.sum(centroids ** 2, axis=1, keepdims=True)
    dist = _s1b(cf, x2, c2)

    idx, pos_x = _s2_topk_gather(dist, x)
    b_idx = (idx.reshape(-1, 1) // (N_TOK // B)).astype(jnp.int32)
    s_tab = jnp.repeat(scale.reshape(B, 3), PATCH * PATCH, axis=1)
    o_tab = jnp.repeat(offset.reshape(B, 3), PATCH * PATCH, axis=1)
    s_tab = jnp.pad(s_tab, ((0, 0), (0, D_PAD - D_PATCH)))
    o_tab = jnp.pad(o_tab, ((0, 0), (0, D_PAD - D_PATCH)))

    out1, f1n, f2n = _s3(pos_x, b_idx, s_tab, o_tab, W_pad,
                         W1_0, W2_0, W1_1, W2_1)
    loss = _s4(f1n, f2n).reshape(())
    return out1, loss


# speedup vs baseline: 2.8816x; 1.1695x over previous
import functools

import jax
import jax.numpy as jnp
from jax import lax
from jax.experimental import pallas as pl
from jax.experimental.pallas import tpu as pltpu
from jax.experimental.pallas import tpu_sc as plsc

B = 16
HW = 320
PATCH = 8
D_PATCH = 192
D_PAD = 256
D_FEAT = 384
N_CLUSTER = 27
N_POS = 128
TEMP = 0.1
N_TOK = 25600
TN = 512


def _s1a_body(x_ref, w_ref, c_ref, f_ref, cf_ref):
    f = jnp.dot(x_ref[...], w_ref[...], preferred_element_type=jnp.float32)
    f_ref[...] = f
    cf_ref[...] = jax.lax.dot_general(c_ref[...], f, (((1,), (1,)), ((), ())),
                                      preferred_element_type=jnp.float32)


def _s1a(x, W_embed, centroids):
    grid = N_TOK // TN
    return pl.pallas_call(
        _s1a_body,
        grid=(grid,),
        in_specs=[
            pl.BlockSpec((TN, D_PATCH), lambda i: (i, 0)),
            pl.BlockSpec((D_PATCH, D_FEAT), lambda i: (0, 0)),
            pl.BlockSpec((N_CLUSTER, D_FEAT), lambda i: (0, 0)),
        ],
        out_specs=[
            pl.BlockSpec((TN, D_FEAT), lambda i: (i, 0)),
            pl.BlockSpec((N_CLUSTER, TN), lambda i: (0, i)),
        ],
        out_shape=[
            jax.ShapeDtypeStruct((N_TOK, D_FEAT), jnp.float32),
            jax.ShapeDtypeStruct((N_CLUSTER, N_TOK), jnp.float32),
        ],
    )(x, W_embed, centroids)


def _s1b_body(cf_ref, x2_ref, c2_ref, dist_ref):
    d2 = c2_ref[...] + x2_ref[...] - 2.0 * cf_ref[...]
    dist_ref[...] = jnp.sqrt(jnp.clip(d2, 0.0))


def _s1b(cf, x2, c2):
    grid = N_TOK // 3200
    return pl.pallas_call(
        _s1b_body,
        grid=(grid,),
        in_specs=[
            pl.BlockSpec((N_CLUSTER, 3200), lambda i: (0, i)),
            pl.BlockSpec((1, 3200), lambda i: (0, i)),
            pl.BlockSpec((N_CLUSTER, 1), lambda i: (0, 0)),
        ],
        out_specs=pl.BlockSpec((N_CLUSTER, 3200), lambda i: (0, i)),
        out_shape=jax.ShapeDtypeStruct((N_CLUSTER, N_TOK), jnp.float32),
    )(cf, x2, c2)


_CHUNK = 256
_NCHUNK = N_TOK // _CHUNK
_L1PAD = 112
_BIG = 1 << 30


def _s2_topk_gather(dist, x):
    mesh = plsc.VectorSubcoreMesh(core_axis_name="c", subcore_axis_name="s")

    @functools.partial(
        pl.kernel, mesh=mesh,
        compiler_params=pltpu.CompilerParams(needs_layout_passes=False,
                                             use_tc_tiling_on_sc=True),
        out_type=[
            jax.ShapeDtypeStruct((N_CLUSTER, N_POS), jnp.int32),
            jax.ShapeDtypeStruct((N_CLUSTER * N_POS, D_PAD), jnp.float32),
        ],
        scratch_types=[
            pltpu.VMEM((N_TOK,), jnp.float32),
            pltpu.VMEM((_L1PAD,), jnp.float32),
            pltpu.VMEM((N_POS,), jnp.int32),
            pltpu.VMEM((N_POS, D_PAD), jnp.float32),
            pltpu.SemaphoreType.DMA,
        ],
    )
    def k(dist_hbm, x_hbm, idx_hbm, posx_hbm, row_v, l1_v, idxb_v, rows_v, sem):
        wid = lax.axis_index("s") * 2 + lax.axis_index("c")

        @pl.when(wid < N_CLUSTER)
        def _():
            iota = lax.iota(jnp.int32, 16)
            pltpu.sync_copy(dist_hbm.at[wid], row_v)

            l1_v[pl.ds(96, 16)] = jnp.where(iota < 4, 0.0, -1.0)

            def build_chunk(c, carry):
                base = c * _CHUNK
                m = row_v[pl.ds(base, 16)]
                for j in range(1, 16):
                    m = jnp.maximum(m, row_v[pl.ds(base + j * 16, 16)])
                cm = jnp.max(m)
                off = (c // 16) * 16
                cur = l1_v[pl.ds(off, 16)]
                l1_v[pl.ds(off, 16)] = jnp.where(iota == (c - off), cm, cur)
                return carry

            lax.fori_loop(0, _NCHUNK, build_chunk, 0)

            def extract(i, carry):
                v = l1_v[pl.ds(0, 16)]
                for g in range(1, 7):
                    v = jnp.maximum(v, l1_v[pl.ds(g * 16, 16)])
                m = jnp.max(v)
                cstar = _BIG
                for g in range(7):
                    lv = l1_v[pl.ds(g * 16, 16)]
                    cand = jnp.where(lv == m, iota + g * 16, _BIG)
                    cstar = jnp.minimum(cstar, jnp.min(cand))
                base = cstar * _CHUNK
                gidx = _BIG
                for j in range(16):
                    vj = row_v[pl.ds(base + j * 16, 16)]
                    cand = jnp.where(vj == m, iota + j * 16, _BIG)
                    gidx = jnp.minimum(gidx, jnp.min(cand))
                gflat = base + gidx
                goff = (i // 16) * 16
                ib = idxb_v[pl.ds(goff, 16)]
                idxb_v[pl.ds(goff, 16)] = jnp.where(iota == (i - goff), gflat, ib)
                ebase = (gflat // 16) * 16
                ev = row_v[pl.ds(ebase, 16)]
                row_v[pl.ds(ebase, 16)] = jnp.where(iota == (gflat - ebase), -1.0, ev)
                m2 = row_v[pl.ds(base, 16)]
                for j in range(1, 16):
                    m2 = jnp.maximum(m2, row_v[pl.ds(base + j * 16, 16)])
                cm2 = jnp.max(m2)
                loff = (cstar // 16) * 16
                lv2 = l1_v[pl.ds(loff, 16)]
                l1_v[pl.ds(loff, 16)] = jnp.where(iota == (cstar - loff), cm2, lv2)
                return carry

            lax.fori_loop(0, N_POS, extract, 0)

            pltpu.async_copy(x_hbm.at[idxb_v], rows_v, sem).wait()
            pltpu.sync_copy(rows_v, posx_hbm.at[pl.ds(wid * N_POS, N_POS)])
            pltpu.sync_copy(idxb_v, idx_hbm.at[wid])

    return k(dist, x)


_TS3 = 384


def _s3_body(px_ref, bi_ref, stab_ref, otab_ref, w_ref, w10_ref, w20_ref,
             w11_ref, w21_ref, out1_ref, f1n_ref, f2n_ref):
    px = px_ref[...]
    f_ori = jnp.dot(px, w_ref[...], preferred_element_type=jnp.float32)
    onehot = (bi_ref[...] == lax.broadcasted_iota(jnp.int32, (1, B), 1)
              ).astype(jnp.float32)
    s_g = jnp.dot(onehot, stab_ref[...], preferred_element_type=jnp.float32)
    o_g = jnp.dot(onehot,
                  jnp.dot(otab_ref[...], w_ref[...],
                          preferred_element_type=jnp.float32),
                  preferred_element_type=jnp.float32)
    f_aug = jnp.dot(px * s_g, w_ref[...], preferred_element_type=jnp.float32) + o_g

    def mlp(h):
        for w1_ref, w2_ref in ((w10_ref, w20_ref), (w11_ref, w21_ref)):
            t = jax.nn.relu(jnp.dot(h, w1_ref[...],
                                    preferred_element_type=jnp.float32))
            h = h + jnp.dot(t, w2_ref[...], preferred_element_type=jnp.float32)
        return h

    h_o = mlp(f_ori)
    h_a = mlp(f_aug)
    out1_ref[...] = h_o
    f1n_ref[...] = h_o / (jnp.sqrt(jnp.sum(h_o * h_o, axis=1, keepdims=True)) + 1e-8)
    f2n_ref[...] = h_a / (jnp.sqrt(jnp.sum(h_a * h_a, axis=1, keepdims=True)) + 1e-8)


def _s3(pos_x, b_idx, s_tab, o_tab, W_pad, W1_0, W2_0, W1_1, W2_1):
    n = N_CLUSTER * N_POS
    grid = n // _TS3
    full = lambda r, c: pl.BlockSpec((r, c), lambda i: (0, 0))
    return pl.pallas_call(
        _s3_body,
        grid=(grid,),
        in_specs=[
            pl.BlockSpec((_TS3, D_PAD), lambda i: (i, 0)),
            pl.BlockSpec((_TS3, 1), lambda i: (i, 0)),
            full(B, D_PAD), full(B, D_PAD), full(D_PAD, D_FEAT),
            full(D_FEAT, D_FEAT), full(D_FEAT, D_FEAT),
            full(D_FEAT, D_FEAT), full(D_FEAT, D_FEAT),
        ],
        out_specs=[
            pl.BlockSpec((_TS3, D_FEAT), lambda i: (i, 0)),
            pl.BlockSpec((_TS3, D_FEAT), lambda i: (i, 0)),
            pl.BlockSpec((_TS3, D_FEAT), lambda i: (i, 0)),
        ],
        out_shape=[
            jax.ShapeDtypeStruct((n, D_FEAT), jnp.float32),
            jax.ShapeDtypeStruct((n, D_FEAT), jnp.float32),
            jax.ShapeDtypeStruct((n, D_FEAT), jnp.float32),
        ],
    )(pos_x, b_idx, s_tab, o_tab, W_pad, W1_0, W2_0, W1_1, W2_1)


def _s4_body(f1_ref, f2_ref, f2d_ref, loss_ref, acc_ref):
    i = pl.program_id(0)
    f1 = f1_ref[...]
    logits = jax.lax.dot_general(
        f1, f2_ref[...], (((1,), (1,)), ((), ())),
        preferred_element_type=jnp.float32) / TEMP
    m = jnp.max(logits, axis=1, keepdims=True)
    lse = m + jnp.log(jnp.sum(jnp.exp(logits - m), axis=1, keepdims=True))
    diag = jnp.sum(f1 * f2d_ref[...], axis=1, keepdims=True) / TEMP
    part = jnp.sum(lse - diag)

    @pl.when(i == 0)
    def _():
        acc_ref[0, 0] = 0.0

    acc_ref[0, 0] += part

    @pl.when(i == pl.num_programs(0) - 1)
    def _():
        loss_ref[...] = jnp.full((1, 1), acc_ref[0, 0] / (N_CLUSTER * N_POS),
                                 jnp.float32)


def _s4(f1n, f2n):
    n = N_CLUSTER * N_POS
    grid = n // _TS3
    return pl.pallas_call(
        _s4_body,
        grid=(grid,),
        in_specs=[
            pl.BlockSpec((_TS3, D_FEAT), lambda i: (i, 0)),
            pl.BlockSpec((n, D_FEAT), lambda i: (0, 0)),
            pl.BlockSpec((_TS3, D_FEAT), lambda i: (i, 0)),
        ],
        out_specs=pl.BlockSpec((1, 1), lambda i: (0, 0)),
        out_shape=jax.ShapeDtypeStruct((1, 1), jnp.float32),
        scratch_shapes=[pltpu.SMEM((1, 1), jnp.float32)],
    )(f1n, f2n, f2n)


def kernel(img, scale, offset, W_embed, centroids, W1_0, W2_0, W1_1, W2_1):
    x = img.reshape(B, 3, HW // PATCH, PATCH, HW // PATCH, PATCH)
    x = x.transpose(0, 2, 4, 1, 3, 5).reshape(N_TOK, D_PATCH)
    x_pad = jnp.pad(x, ((0, 0), (0, D_PAD - D_PATCH)))
    W_pad = jnp.pad(W_embed, ((0, D_PAD - D_PATCH), (0, 0)))

    f = x @ W_embed
    x2 = jnp.sum(f ** 2, axis=1)[None, :]
    c2 = jnp.sum(centroids ** 2, axis=1, keepdims=True)
    dist = jnp.sqrt(jnp.clip(c2 + x2 - 2.0 * (centroids @ f.T), 0.0))

    idx, pos_x = _s2_topk_gather(dist, x_pad)
    b_idx = (idx.reshape(-1, 1) // (N_TOK // B)).astype(jnp.int32)
    s_tab = jnp.repeat(scale.reshape(B, 3), PATCH * PATCH, axis=1)
    o_tab = jnp.repeat(offset.reshape(B, 3), PATCH * PATCH, axis=1)
    s_tab = jnp.pad(s_tab, ((0, 0), (0, D_PAD - D_PATCH)))
    o_tab = jnp.pad(o_tab, ((0, 0), (0, D_PAD - D_PATCH)))

    out1, f1n, f2n = _s3(pos_x, b_idx, s_tab, o_tab, W_pad,
                         W1_0, W2_0, W1_1, W2_1)
    loss = _s4(f1n, f2n).reshape(())
    return out1, loss
